# baseline (speedup 1.0000x reference)
import functools

import jax
import jax.numpy as jnp
from jax import lax
from jax.experimental import pallas as pl
from jax.experimental.pallas import tpu as pltpu
from jax.experimental.pallas import tpu_sc as plsc

_N_H = 8
_N = 64
_NUM_CLS = 117
_NUM_OBJ = 80
_HUMAN = 49
_DIM = 300
_PAIRS = _N_H * _N - _N_H
_DEPTH = 16


def _l2n(x):
    return x / jnp.maximum(jnp.sqrt(jnp.sum(x * x, axis=-1, keepdims=True)),
                           1e-12)


def _prep_body(lab_ref, ent_ref, rel_ref, nv_ref, oh_ref,
               hp_o, rn_o, wn_o, tp_o, s_o):
    lab = jnp.where(lax.broadcasted_iota(jnp.int32, (_N, 1), 0) < _N_H,
                    _HUMAN, lab_ref[...])
    oh64 = (lab == lax.broadcasted_iota(jnp.int32, (_N, _NUM_OBJ), 1)
            ).astype(jnp.float32)
    ent = ent_ref[...]
    tn = _l2n(jnp.dot(oh64, ent, preferred_element_type=jnp.float32))
    hn = _l2n(ent[_HUMAN:_HUMAN + 1, :])
    wn = _l2n(nv_ref[...])
    rn = _l2n(rel_ref[...])
    hp = hn - jnp.sum(hn * wn, axis=-1, keepdims=True) * wn
    hp_o[...] = hp
    rn_o[...] = rn
    wn_o[...] = wn
    d = lax.dot_general(tn, wn, (((1,), (1,)), ((), ())),
                        preferred_element_type=jnp.float32)
    tp = tn[:, None, :] - d[:, :, None] * wn[None, :, :]
    tp_o[...] = tp
    diff = (hp + rn)[None, :, :] - tp
    s = jnp.sqrt(jnp.sum(diff * diff, axis=-1))
    s_o[...] = jnp.dot(oh_ref[...], s, preferred_element_type=jnp.float32)


def _prep(box_labels, ent_emb, rel_emb, norm_vec, oh504):
    small = jax.ShapeDtypeStruct((_NUM_CLS, _DIM), jnp.float32)
    return pl.pallas_call(
        _prep_body,
        out_shape=(small, small, small,
                   jax.ShapeDtypeStruct((_N, _NUM_CLS, _DIM), jnp.float32),
                   jax.ShapeDtypeStruct((_PAIRS, _NUM_CLS), jnp.float32)),
    )(box_labels.reshape(_N, 1), ent_emb, rel_emb, norm_vec, oh504)


_BIG = jax.ShapeDtypeStruct((_PAIRS, _NUM_CLS, _DIM), jnp.float32)


def _t_scatter(buf, y, t_out, sem):
    def fire(x, c):
        j = jnp.where(y < x, y, y - 1)

        @pl.when(x != y)
        def _():
            pltpu.async_copy(buf, t_out.at[63 * x + j], sem)
        return c

    lax.fori_loop(0, _N_H, fire, 0)

    def drain(x, c):
        j = jnp.where(y < x, y, y - 1)

        @pl.when(x != y)
        def _():
            pltpu.make_async_copy(buf, t_out.at[63 * x + j], sem).wait()
        return c

    lax.fori_loop(0, _N_H, drain, 0)


@functools.partial(
    pl.kernel,
    out_type=[_BIG, _BIG, _BIG, _BIG],
    mesh=plsc.VectorSubcoreMesh(core_axis_name="c", subcore_axis_name="s"),
    compiler_params=pltpu.CompilerParams(use_tc_tiling_on_sc=True),
    scratch_types=[
        pltpu.VMEM((_NUM_CLS, _DIM), jnp.float32),
        pltpu.VMEM((_NUM_CLS, _DIM), jnp.float32),
        pltpu.SemaphoreType.DMA,
        pltpu.SemaphoreType.DMA,
    ],
)
def _expand(hp_hbm, rn_hbm, wn_hbm, tp_hbm,
            h_out, r_out, w_out, t_out, tab_v, aux_v, sem, lsem):
    cid = lax.axis_index("c")
    sid = lax.axis_index("s")
    wid = sid * 2 + cid
    y0 = 2 * wid
    y1 = y0 + 1

    pltpu.async_copy(tp_hbm.at[y0], aux_v, lsem)

    outs = (h_out, r_out, w_out)
    tabs = (hp_hbm, rn_hbm, wn_hbm)
    bases = (0, 11, 22)
    counts = (11, 11, 10)

    for m in range(3):
        @pl.when(jnp.logical_and(wid >= bases[m], wid < bases[m] + counts[m]))
        def _bcast(m=m):
            pltpu.sync_copy(tabs[m], tab_v)
            idx = wid - bases[m]
            ra = idx * _PAIRS // counts[m]
            rb = (idx + 1) * _PAIRS // counts[m]

            def fire(r, c):
                pltpu.async_copy(tab_v, outs[m].at[r], sem)

                @pl.when(r - ra >= _DEPTH)
                def _():
                    pltpu.make_async_copy(tab_v, outs[m].at[r - _DEPTH],
                                          sem).wait()
                return c

            lax.fori_loop(ra, rb, fire, 0)

            def drain(r, c):
                pltpu.make_async_copy(tab_v, outs[m].at[r], sem).wait()
                return c

            lax.fori_loop(jnp.maximum(ra, rb - _DEPTH), rb, drain, 0)

    pltpu.make_async_copy(tp_hbm.at[y0], aux_v, lsem).wait()
    _t_scatter(aux_v, y0, t_out, sem)
    pltpu.sync_copy(tp_hbm.at[y1], tab_v)
    _t_scatter(tab_v, y1, t_out, sem)


def _static_onehot():
    import numpy as np
    ys = np.array([j + (1 if j >= x else 0)
                   for x in range(_N_H) for j in range(_N - 1)], np.int32)
    return (ys[:, None] == np.arange(_N)[None, :]).astype(np.float32)


_OH504 = _static_onehot()


def kernel(box_labels, ent_emb, rel_emb, norm_vec):
    hp, rn, wn, tp, scores = _prep(box_labels, ent_emb, rel_emb, norm_vec,
                                   jnp.asarray(_OH504))
    h_keep, r_keep, w_keep, t_keep = _expand(hp, rn, wn, tp)
    return (h_keep, r_keep, w_keep, t_keep, scores)


# speedup vs baseline: 4.7318x; 4.7318x over previous
import jax
import jax.numpy as jnp
from jax import lax
from jax.experimental import pallas as pl
from jax.experimental.pallas import tpu as pltpu

_N_H = 8
_N = 64
_NUM_CLS = 117
_NUM_OBJ = 80
_HUMAN = 49
_DIM = 300
_PAIRS = _N_H * _N - _N_H
_CB = 8
_STEPS = -(-_NUM_CLS // _CB)


def _l2n(x):
    return x / jnp.maximum(jnp.sqrt(jnp.sum(x * x, axis=-1, keepdims=True)),
                           1e-12)


def _compute_tn(lab_ref, ent):
    lab = jnp.where(lax.broadcasted_iota(jnp.int32, (_N, 1), 0) < _N_H,
                    _HUMAN, lab_ref[...])
    oh = (lab == lax.broadcasted_iota(jnp.int32, (_N, _NUM_OBJ), 1)
          ).astype(jnp.float32)
    return _l2n(jnp.dot(oh, ent, preferred_element_type=jnp.float32))


def _main_body(lab_ref, ent_ref, rel_ref, nv_ref, oh_ref,
               h_o, r_o, w_o, t_o, tn_s):
    i = pl.program_id(0)

    @pl.when(i == 0)
    def _prep():
        tn_s[...] = _compute_tn(lab_ref, ent_ref[...])

    wn = _l2n(nv_ref[...])
    rn = _l2n(rel_ref[...])
    hn = _l2n(ent_ref[_HUMAN:_HUMAN + 1, :])
    hp = hn - jnp.sum(hn * wn, axis=-1, keepdims=True) * wn
    blk = (_CB, _DIM, _PAIRS)
    h_o[...] = jnp.broadcast_to(hp[:, :, None], blk)
    r_o[...] = jnp.broadcast_to(rn[:, :, None], blk)
    w_o[...] = jnp.broadcast_to(wn[:, :, None], blk)
    tn = tn_s[...]
    d = lax.dot_general(tn, wn, (((1,), (1,)), ((), ())),
                        preferred_element_type=jnp.float32)
    oh = oh_ref[...]
    for c in range(_CB):
        tp_c = tn - d[:, c:c + 1] * wn[c:c + 1, :]
        t_o[c] = lax.dot_general(tp_c, oh, (((0,), (0,)), ((), ())),
                                 preferred_element_type=jnp.float32)


def _scores_body(lab_ref, ent_ref, rel_ref, nv_ref, oh_ref, s_o):
    ent = ent_ref[...]
    tn = _compute_tn(lab_ref, ent)
    wn = _l2n(nv_ref[...])
    rn = _l2n(rel_ref[...])
    hn = _l2n(ent[_HUMAN:_HUMAN + 1, :])
    hp = hn - jnp.sum(hn * wn, axis=-1, keepdims=True) * wn
    a = hp + rn
    big = lax.dot_general(a, tn, (((1,), (1,)), ((), ())),
                          preferred_element_type=jnp.float32)
    d_t = lax.dot_general(wn, tn, (((1,), (1,)), ((), ())),
                          preferred_element_type=jnp.float32)
    aw = jnp.sum(a * wn, axis=-1, keepdims=True)
    na2 = jnp.sum(a * a, axis=-1, keepdims=True)
    nt2 = lax.dot_general(jnp.ones((1, _DIM), jnp.float32), tn * tn,
                          (((1,), (1,)), ((), ())),
                          preferred_element_type=jnp.float32)
    s2 = na2 + nt2 - 2.0 * big + (2.0 * aw - d_t) * d_t
    s = jnp.sqrt(jnp.maximum(s2, 0.0))
    s_o[...] = jnp.dot(s, oh_ref[...],
                       preferred_element_type=jnp.float32)


def _static_onehot():
    import numpy as np
    ys = np.array([j + (1 if j >= x else 0)
                   for x in range(_N_H) for j in range(_N - 1)], np.int32)
    return (np.arange(_N)[:, None] == ys[None, :]).astype(np.float32)


_OH64 = _static_onehot()


def kernel(box_labels, ent_emb, rel_emb, norm_vec):
    lab2d = box_labels.reshape(_N, 1)
    oh = jnp.asarray(_OH64)
    big_t = jax.ShapeDtypeStruct((_NUM_CLS, _DIM, _PAIRS), jnp.float32)
    const2 = pl.BlockSpec((_N, 1), lambda i: (0, 0))
    const_e = pl.BlockSpec((_NUM_OBJ, _DIM), lambda i: (0, 0))
    cls_blk = pl.BlockSpec((_CB, _DIM), lambda i: (i, 0))
    const_oh = pl.BlockSpec((_N, _PAIRS), lambda i: (0, 0))
    out_blk = pl.BlockSpec((_CB, _DIM, _PAIRS), lambda i: (i, 0, 0))
    hT, rT, wT, tT = pl.pallas_call(
        _main_body,
        grid=(_STEPS,),
        in_specs=[const2, const_e, cls_blk, cls_blk, const_oh],
        out_specs=(out_blk, out_blk, out_blk, out_blk),
        out_shape=(big_t, big_t, big_t, big_t),
        scratch_shapes=[pltpu.VMEM((_N, _DIM), jnp.float32)],
    )(lab2d, ent_emb, rel_emb, norm_vec, oh)
    sT = pl.pallas_call(
        _scores_body,
        out_shape=jax.ShapeDtypeStruct((_NUM_CLS, _PAIRS), jnp.float32),
    )(lab2d, ent_emb, rel_emb, norm_vec, oh)
    perm = (2, 0, 1)
    return (jnp.transpose(hT, perm), jnp.transpose(rT, perm),
            jnp.transpose(wT, perm), jnp.transpose(tT, perm),
            jnp.transpose(sT, (1, 0)))


# speedup vs baseline: 4.9005x; 1.0356x over previous
import jax
import jax.numpy as jnp
from jax import lax
from jax.experimental import pallas as pl
from jax.experimental.pallas import tpu as pltpu

_N_H = 8
_N = 64
_NUM_CLS = 117
_NUM_OBJ = 80
_HUMAN = 49
_DIM = 300
_PAIRS = _N_H * _N - _N_H
_CB = 8
_STEPS = -(-_NUM_CLS // _CB)
_CPAD = _STEPS * _CB


def _l2n(x):
    return x / jnp.maximum(jnp.sqrt(jnp.sum(x * x, axis=-1, keepdims=True)),
                           1e-12)


def _body(lab_ref, ent_ref, relt_ref, nvt_ref, oh_ref,
          h_o, r_o, w_o, t_o, s_o, tn_s, hp_s, rn_s, wn_s):
    i = pl.program_id(0)

    @pl.when(i == 0)
    def _prep():
        ent = ent_ref[...]
        lab = jnp.where(lax.broadcasted_iota(jnp.int32, (1, _N), 1) < _N_H,
                        _HUMAN, lab_ref[...])
        oh_t = (lab == lax.broadcasted_iota(jnp.int32, (_NUM_OBJ, _N), 0)
                ).astype(jnp.float32)
        tn = _l2n(lax.dot_general(oh_t, ent, (((0,), (0,)), ((), ())),
                                  preferred_element_type=jnp.float32))
        tn_s[...] = tn
        eye = (lax.broadcasted_iota(jnp.int32, (_CPAD, _NUM_CLS), 0) ==
               lax.broadcasted_iota(jnp.int32, (_CPAD, _NUM_CLS), 1)
               ).astype(jnp.float32)
        rel = lax.dot_general(eye, relt_ref[...], (((1,), (1,)), ((), ())),
                              preferred_element_type=jnp.float32)
        nv = lax.dot_general(eye, nvt_ref[...], (((1,), (1,)), ((), ())),
                             preferred_element_type=jnp.float32)
        wn = _l2n(nv)
        rn = _l2n(rel)
        hn = _l2n(ent[_HUMAN:_HUMAN + 1, :])
        hp = hn - jnp.sum(hn * wn, axis=-1, keepdims=True) * wn
        wn_s[...] = wn
        rn_s[...] = rn
        hp_s[...] = hp
        a = (hp + rn)[:_NUM_CLS, :]
        wn_c = wn[:_NUM_CLS, :]
        big = lax.dot_general(a, tn, (((1,), (1,)), ((), ())),
                              preferred_element_type=jnp.float32)
        d_t = lax.dot_general(wn_c, tn, (((1,), (1,)), ((), ())),
                              preferred_element_type=jnp.float32)
        aw = jnp.sum(a * wn_c, axis=-1, keepdims=True)
        na2 = jnp.sum(a * a, axis=-1, keepdims=True)
        nt2 = lax.dot_general(jnp.ones((1, _DIM), jnp.float32), tn * tn,
                              (((1,), (1,)), ((), ())),
                              preferred_element_type=jnp.float32)
        s2 = na2 + nt2 - 2.0 * big + (2.0 * aw - d_t) * d_t
        s = jnp.sqrt(jnp.maximum(s2, 0.0))
        s_o[...] = jnp.dot(s, oh_ref[...],
                           preferred_element_type=jnp.float32)

    sl = pl.ds(i * _CB, _CB)
    wn = wn_s[sl, :]
    rn = rn_s[sl, :]
    hp = hp_s[sl, :]
    blk = (_CB, _DIM, _PAIRS)
    h_o[...] = jnp.broadcast_to(hp[:, :, None], blk)
    r_o[...] = jnp.broadcast_to(rn[:, :, None], blk)
    w_o[...] = jnp.broadcast_to(wn[:, :, None], blk)
    tn = tn_s[...]
    d = lax.dot_general(tn, wn, (((1,), (1,)), ((), ())),
                        preferred_element_type=jnp.float32)
    oh = oh_ref[...]
    for c in range(_CB):
        tp_c = tn - d[:, c:c + 1] * wn[c:c + 1, :]
        t_o[c] = lax.dot_general(tp_c, oh, (((0,), (0,)), ((), ())),
                                 preferred_element_type=jnp.float32)


def _static_onehot():
    import numpy as np
    ys = np.array([j + (1 if j >= x else 0)
                   for x in range(_N_H) for j in range(_N - 1)], np.int32)
    return (np.arange(_N)[:, None] == ys[None, :]).astype(np.float32)


_OH64 = _static_onehot()


def kernel(box_labels, ent_emb, rel_emb, norm_vec):
    big_t = jax.ShapeDtypeStruct((_NUM_CLS, _DIM, _PAIRS), jnp.float32)
    const = lambda shape: pl.BlockSpec(shape, lambda i: (0,) * len(shape))
    out_blk = pl.BlockSpec((_CB, _DIM, _PAIRS), lambda i: (i, 0, 0))
    hT, rT, wT, tT, sT = pl.pallas_call(
        _body,
        grid=(_STEPS,),
        in_specs=[const((1, _N)), const((_NUM_OBJ, _DIM)),
                  const((_DIM, _NUM_CLS)), const((_DIM, _NUM_CLS)),
                  const((_N, _PAIRS))],
        out_specs=(out_blk, out_blk, out_blk, out_blk,
                   const((_NUM_CLS, _PAIRS))),
        out_shape=(big_t, big_t, big_t, big_t,
                   jax.ShapeDtypeStruct((_NUM_CLS, _PAIRS), jnp.float32)),
        scratch_shapes=[pltpu.VMEM((_N, _DIM), jnp.float32),
                        pltpu.VMEM((_CPAD, _DIM), jnp.float32),
                        pltpu.VMEM((_CPAD, _DIM), jnp.float32),
                        pltpu.VMEM((_CPAD, _DIM), jnp.float32)],
    )(box_labels.reshape(1, _N), ent_emb, rel_emb.T, norm_vec.T,
      jnp.asarray(_OH64))
    perm = (2, 0, 1)
    return (jnp.transpose(hT, perm), jnp.transpose(rT, perm),
            jnp.transpose(wT, perm), jnp.transpose(tT, perm),
            jnp.transpose(sT, (1, 0)))
